# baseline (speedup 1.0000x reference)
import jax
import jax.numpy as jnp
from jax import lax
from jax.experimental import pallas as pl
from jax.experimental.pallas import tpu as pltpu
from jax.experimental.pallas import tpu_sc as plsc

B = 16384
D = 64
NC = 2
NS = 16
L = 16
NW = NC * NS
BPW = B // NW
CHUNK = 128
NCHUNK = BPW // CHUNK
NG = CHUNK // L


def _mf_body(users_hbm, items_hbm, utab_hbm, itab_hbm, out_hbm,
             uidx, iidx, ubid, ibid, ubuf, ibuf, outv, sem0, sem1):
    wid = lax.axis_index("s") * NC + lax.axis_index("c")
    base = wid * BPW

    pltpu.sync_copy(users_hbm.at[pl.ds(base, BPW)], uidx)
    pltpu.sync_copy(items_hbm.at[pl.ds(base, BPW)], iidx)

    def shift_group(k, carry):
        s = k * L
        ubid[pl.ds(s, L)] = uidx[pl.ds(s, L)] >> 1
        ibid[pl.ds(s, L)] = iidx[pl.ds(s, L)] >> 1
        return carry

    lax.fori_loop(0, BPW // L, shift_group, 0)

    sems = (sem0, sem1)

    def issue(c, slot, sem):
        pltpu.async_copy(
            utab_hbm.at[ubid.at[pl.ds(c * CHUNK, CHUNK)]], ubuf.at[slot], sem)
        pltpu.async_copy(
            itab_hbm.at[ibid.at[pl.ds(c * CHUNK, CHUNK)]], ibuf.at[slot], sem)

    def finish(c, slot, sem):
        pltpu.make_async_copy(
            utab_hbm.at[ubid.at[pl.ds(0, CHUNK)]], ubuf.at[slot], sem).wait()
        pltpu.make_async_copy(
            itab_hbm.at[ibid.at[pl.ds(0, CHUNK)]], ibuf.at[slot], sem).wait()
        for g in range(NG):
            lanes = g * L + lax.iota(jnp.int32, L)
            hu = (uidx[pl.ds(c * CHUNK + g * L, L)] & 1) << 6
            hi = (iidx[pl.ds(c * CHUNK + g * L, L)] & 1) << 6
            acc = jnp.zeros((L,), jnp.float32)
            for d in range(D):
                u = plsc.load_gather(ubuf.at[slot], [lanes, hu + d])
                v = plsc.load_gather(ibuf.at[slot], [lanes, hi + d])
                acc = acc + u * v
            outv[pl.ds(c * CHUNK + g * L, L)] = acc

    issue(0, 0, sems[0])

    def step(t, carry):
        issue(2 * t + 1, 1, sems[1])
        finish(2 * t, 0, sems[0])

        def more():
            issue(2 * t + 2, 0, sems[0])
            return 0

        lax.cond(t + 1 < NCHUNK // 2, more, lambda: 0)
        finish(2 * t + 1, 1, sems[1])
        return carry

    lax.fori_loop(0, NCHUNK // 2, step, 0)

    pltpu.sync_copy(outv, out_hbm.at[pl.ds(base, BPW)])


def kernel(users, items, users_table, items_table):
    nu, ni = users_table.shape[0], items_table.shape[0]
    ut2 = users_table.reshape(nu // 2, 2 * D)
    it2 = items_table.reshape(ni // 2, 2 * D)
    mesh = plsc.VectorSubcoreMesh(core_axis_name="c", subcore_axis_name="s")
    run = pl.kernel(
        _mf_body,
        out_type=jax.ShapeDtypeStruct((B,), jnp.float32),
        mesh=mesh,
        compiler_params=pltpu.CompilerParams(needs_layout_passes=False),
        scratch_types=[
            pltpu.VMEM((BPW,), jnp.int32),
            pltpu.VMEM((BPW,), jnp.int32),
            pltpu.VMEM((BPW,), jnp.int32),
            pltpu.VMEM((BPW,), jnp.int32),
            pltpu.VMEM((2, CHUNK, 2 * D), jnp.float32),
            pltpu.VMEM((2, CHUNK, 2 * D), jnp.float32),
            pltpu.VMEM((BPW,), jnp.float32),
            pltpu.SemaphoreType.DMA,
            pltpu.SemaphoreType.DMA,
        ],
    )
    return run(users.astype(jnp.int32), items.astype(jnp.int32), ut2, it2)
